# Initial kernel scaffold; baseline (speedup 1.0000x reference)
import jax
import jax.numpy as jnp
from jax.experimental import pallas as pl


def kernel(z_e, codebooks):
    raise NotImplementedError("write your pallas kernel here")


# speedup vs baseline: 1.0480x; 1.0480x over previous
import dataclasses
import functools

import jax
import jax.numpy as jnp
from jax import lax
from jax.experimental import pallas as pl
from jax.experimental.pallas import tpu as pltpu
from jax.experimental.pallas import tpu_sc as plsc

_L = 4
_K = 8192
_D = 256
_B = 16384
_BETA = 0.25
_ENTROPY_REG = 0.0001

_TB = 1024
_TK = 2048
_NB = _B // _TB
_NK = _K // _TK

_NC = 2
_NS = 16
_LANES = 16
_NW = _NC * _NS
_BW = _B // _NW
_CH = 128
_NCHUNK = _BW // _CH


def _argmin_body(has_q, precision, r_ref, q_ref, c_ref, idx_ref, rout_ref,
                 minv_ref, mini_ref, rnorm_ref):
    kt = pl.program_id(1)

    @pl.when(kt == 0)
    def _():
        if has_q:
            rout_ref[...] = r_ref[...] - q_ref[...]
        else:
            rout_ref[...] = r_ref[...]
        rr = rout_ref[...]
        rnorm_ref[...] = jnp.sum(rr * rr, axis=1, keepdims=True)

    R = rout_ref[...]
    C = c_ref[...]
    c_norm = jnp.sum(C * C, axis=1)
    del precision
    dot = lax.dot_general(R.astype(jnp.bfloat16), C.astype(jnp.bfloat16),
                          (((1,), (1,)), ((), ())),
                          preferred_element_type=jnp.float32)
    dists = rnorm_ref[...] - 2.0 * dot + c_norm[None, :]
    lmin = jnp.min(dists, axis=1)
    kidx = lax.broadcasted_iota(jnp.int32, (_TB, _TK), 1)
    lidx = jnp.min(jnp.where(dists == lmin[:, None], kidx, _K), axis=1)
    lidx = lidx + kt * _TK

    lminb = lmin.astype(jnp.bfloat16).astype(jnp.float32)

    @pl.when(kt == 0)
    def _():
        minv_ref[0, :] = lminb
        mini_ref[0, :] = lidx

    @pl.when(kt > 0)
    def _():
        better = lmin < minv_ref[0, :]
        minv_ref[0, :] = jnp.where(better, lminb, minv_ref[0, :])
        mini_ref[0, :] = jnp.where(better, lidx, mini_ref[0, :])

    @pl.when(kt == _NK - 1)
    def _():
        idx_ref[0, 0, :] = mini_ref[0, :]


def _tc_argmin(resid, q_prev, C, has_q, precision=None):
    body = functools.partial(_argmin_body, has_q, precision)
    grid = (_NB, _NK)
    r_spec = pl.BlockSpec((_TB, _D), lambda bt, kt: (bt, 0))
    q_spec = pl.BlockSpec((_TB, _D), lambda bt, kt: (bt, 0))
    c_spec = pl.BlockSpec((_TK, _D), lambda bt, kt: (kt, 0))
    idx_spec = pl.BlockSpec((1, 1, _TB), lambda bt, kt: (bt, 0, 0))
    rout_spec = pl.BlockSpec((_TB, _D), lambda bt, kt: (bt, 0))
    idx3, rout = pl.pallas_call(
        body,
        grid=grid,
        in_specs=[r_spec, q_spec, c_spec],
        out_specs=[idx_spec, rout_spec],
        out_shape=[
            jax.ShapeDtypeStruct((_NB, 1, _TB), jnp.int32),
            jax.ShapeDtypeStruct((_B, _D), jnp.float32),
        ],
        scratch_shapes=[
            pltpu.VMEM((1, _TB), jnp.float32),
            pltpu.VMEM((1, _TB), jnp.int32),
            pltpu.VMEM((_TB, 1), jnp.float32),
        ],
    )(resid, q_prev, C)
    return idx3.reshape(_B), rout


def _sc_body(c_hbm, idx_hbm, out_hbm, cnt_hbm, idx_v, rows_v, cnt_v, sem1):
    cid = lax.axis_index("c")
    sid = lax.axis_index("s")
    wid = sid * _NC + cid

    @pl.loop(0, _K, step=_LANES)
    def _(i):
        cnt_v[pl.ds(i, _LANES)] = jnp.zeros((_LANES,), jnp.float32)

    pltpu.sync_copy(idx_hbm.at[wid], idx_v)
    iota = lax.iota(jnp.int32, _LANES)

    @pl.loop(0, _NCHUNK)
    def _(j):
        base = wid * _BW + j * _CH
        pltpu.async_copy(c_hbm.at[idx_v.at[j]], rows_v, sem1).wait()
        pltpu.sync_copy(rows_v, out_hbm.at[pl.ds(base, _CH)])
        @pl.loop(0, _CH, step=_LANES)
        def _(o):
            v = idx_v[j, pl.ds(o, _LANES)]
            cnt = jnp.ones((_LANES,), jnp.float32)
            dup = iota < 0
            for s in range(1, _LANES):
                perm = iota + s
                perm = jnp.where(perm >= _LANES, perm - _LANES, perm)
                w = v.at[perm].get(mode="promise_in_bounds")
                eq = w == v
                cnt = cnt + jnp.where(eq, 1.0, 0.0)
                dup = jnp.logical_or(
                    dup, jnp.logical_and(eq, iota >= _LANES - s))
            plsc.addupdate_scatter(cnt_v, [v], cnt,
                                   mask=jnp.logical_not(dup))

    pltpu.sync_copy(cnt_v, cnt_hbm.at[wid])


def _sc_gather(C, idx):
    idx3 = idx.reshape(_NW, _NCHUNK, _CH)
    mesh = plsc.VectorSubcoreMesh(core_axis_name="c", subcore_axis_name="s")
    cp = pltpu.CompilerParams()
    if "needs_layout_passes" in pltpu.CompilerParams.__dataclass_fields__:
        cp = dataclasses.replace(cp, needs_layout_passes=False)
    kern = pl.kernel(
        _sc_body,
        mesh=mesh,
        compiler_params=cp,
        out_type=[
            jax.ShapeDtypeStruct((_B, _D), jnp.float32),
            jax.ShapeDtypeStruct((_NW, _K), jnp.float32),
        ],
        scratch_types=[
            pltpu.VMEM((_NCHUNK, _CH), jnp.int32),
            pltpu.VMEM((_CH, _D), jnp.float32),
            pltpu.VMEM((_K,), jnp.float32),
            pltpu.SemaphoreType.DMA,
        ],
    )
    return kern(C, idx3)


def _final_body(ze_ref, r_ref, q_ref, cnt_ref, zq_ref, commit_ref, reg_ref,
                acc_ref):
    bt = pl.program_id(0)

    @pl.when(bt == 0)
    def _():
        acc_ref[0] = jnp.float32(0.0)

    r4 = r_ref[...] - q_ref[...]
    zq_ref[...] = ze_ref[...] - r4
    acc_ref[0] += jnp.sum(r4 * r4)

    @pl.when(bt == _NB - 1)
    def _():
        commit = _BETA * acc_ref[0] / jnp.float32(_B * _D)
        commit_ref[...] = jnp.broadcast_to(commit, (1, 1))
        ent_sum = jnp.float32(0.0)
        for l in range(_L):
            cl = jnp.sum(cnt_ref[l], axis=0)
            p = cl / jnp.maximum(jnp.sum(cl), 1e-09)
            ent = -jnp.sum(p * jnp.log(jnp.maximum(p, 1e-09)))
            ent_sum = ent_sum + ent
        reg_ref[...] = jnp.broadcast_to(-_ENTROPY_REG * (ent_sum / _L), (1, 1))


def _tc_final(z_e, r3, q4, counts):
    grid = (_NB,)
    bspec = pl.BlockSpec((_TB, _D), lambda bt: (bt, 0))
    cnt_spec = pl.BlockSpec((_L, _NW, _K), lambda bt: (0, 0, 0))
    scal_spec = pl.BlockSpec((1, 1), lambda bt: (0, 0))
    zq, commit, reg = pl.pallas_call(
        _final_body,
        grid=grid,
        in_specs=[bspec, bspec, bspec, cnt_spec],
        out_specs=[bspec, scal_spec, scal_spec],
        out_shape=[
            jax.ShapeDtypeStruct((_B, _D), jnp.float32),
            jax.ShapeDtypeStruct((1, 1), jnp.float32),
            jax.ShapeDtypeStruct((1, 1), jnp.float32),
        ],
        scratch_shapes=[pltpu.SMEM((1,), jnp.float32)],
    )(z_e, r3, q4, counts)
    return zq, commit.reshape(()), reg.reshape(())


def kernel(z_e, codebooks):
    z_e = z_e.astype(jnp.float32)
    resid = z_e
    q = z_e
    ids = []
    counts = []
    for l in range(_L):
        idx_l, resid = _tc_argmin(resid, q, codebooks[l], has_q=(l > 0))
        q, cnt_l = _sc_gather(codebooks[l], idx_l)
        ids.append(idx_l)
        counts.append(cnt_l)
    cnt_all = jnp.stack(counts, axis=0)
    z_q, commit, reg = _tc_final(z_e, resid, q, cnt_all)
    ids_arr = jnp.stack(ids, axis=0)
    return (z_q, ids_arr, commit, reg)
